# Initial kernel scaffold; baseline (speedup 1.0000x reference)
import jax
import jax.numpy as jnp
from jax.experimental import pallas as pl


def kernel(x, weight):
    raise NotImplementedError("write your pallas kernel here")


# speedup vs baseline: 1.0445x; 1.0445x over previous
import functools

import jax
import jax.numpy as jnp
from jax import lax
from jax.experimental import pallas as pl
from jax.experimental.pallas import tpu as pltpu
from jax.experimental.pallas import tpu_sc as plsc

EMBED_DIM = 32
CHUNK = 128
NBUF = 2


@functools.cache
def _make_kernel(n_flat: int):
    info = plsc.get_sparse_core_info()
    num_workers = info.num_cores * info.num_subcores
    b_per_w = n_flat // num_workers
    steps = b_per_w // CHUNK
    mesh = plsc.VectorSubcoreMesh(core_axis_name="c", subcore_axis_name="s")

    @functools.partial(
        pl.kernel,
        mesh=mesh,
        out_type=jax.ShapeDtypeStruct((n_flat, EMBED_DIM), jnp.float32),
        scratch_types=[
            pltpu.VMEM((steps, CHUNK), jnp.int32),
            *[pltpu.VMEM((CHUNK, EMBED_DIM), jnp.float32) for _ in range(NBUF)],
            pltpu.SemaphoreType.DMA,
            pltpu.SemaphoreType.DMA,
        ],
        compiler_params=pltpu.CompilerParams(use_tc_tiling_on_sc=False),
    )
    def emb_kernel(idx_hbm, table_hbm, out_hbm, idx_v, *rest):
        bufs = rest[:NBUF]
        gsem = rest[NBUF]
        osem = rest[NBUF + 1]
        wid = lax.axis_index("s") * info.num_cores + lax.axis_index("c")
        base = wid * b_per_w
        pltpu.sync_copy(idx_hbm.at[wid], idx_v)

        def outer(jo, _):
            for b in range(NBUF):
                j = jo * NBUF + b
                @pl.when(jo > 0)
                def _wait_out():
                    pltpu.make_async_copy(
                        bufs[b], out_hbm.at[pl.ds(base, CHUNK)], osem
                    ).wait()

                pltpu.async_copy(table_hbm.at[idx_v.at[j]], bufs[b], gsem).wait()
                pltpu.async_copy(
                    bufs[b], out_hbm.at[pl.ds(base + j * CHUNK, CHUNK)], osem
                )
            return 0

        lax.fori_loop(0, steps // NBUF, outer, 0)
        for b in range(NBUF):
            pltpu.make_async_copy(
                bufs[b], out_hbm.at[pl.ds(base, CHUNK)], osem
            ).wait()

    return emb_kernel, num_workers, steps


def kernel(x, weight):
    batch, hist = x.shape
    n_flat = batch * hist
    emb, num_workers, steps = _make_kernel(n_flat)
    idx = x.reshape(num_workers, steps, CHUNK)
    out = emb(idx, weight)
    return out.reshape(batch, hist, EMBED_DIM)


# speedup vs baseline: 1.1130x; 1.0656x over previous
import functools

import jax
import jax.numpy as jnp
from jax import lax
from jax.experimental import pallas as pl
from jax.experimental.pallas import tpu as pltpu
from jax.experimental.pallas import tpu_sc as plsc

EMBED_DIM = 32
CHUNK = 128
NBUF = 8
PRE = 4


@functools.cache
def _make_kernel(n_flat: int):
    info = plsc.get_sparse_core_info()
    num_workers = info.num_cores * info.num_subcores
    b_per_w = n_flat // num_workers
    steps = b_per_w // CHUNK
    mesh = plsc.VectorSubcoreMesh(core_axis_name="c", subcore_axis_name="s")

    @functools.partial(
        pl.kernel,
        mesh=mesh,
        out_type=jax.ShapeDtypeStruct((n_flat, EMBED_DIM), jnp.float32),
        scratch_types=[
            pltpu.VMEM((steps, CHUNK), jnp.int32),
            *[pltpu.VMEM((CHUNK, EMBED_DIM), jnp.float32) for _ in range(NBUF)],
            pltpu.SemaphoreType.DMA,
            pltpu.SemaphoreType.DMA,
        ],
        compiler_params=pltpu.CompilerParams(use_tc_tiling_on_sc=False),
    )
    def emb_kernel(idx_hbm, table_hbm, out_hbm, idx_v, *rest):
        bufs = rest[:NBUF]
        gsem = rest[NBUF]
        osem = rest[NBUF + 1]
        wid = lax.axis_index("s") * info.num_cores + lax.axis_index("c")
        base = wid * b_per_w
        pltpu.sync_copy(idx_hbm.at[wid], idx_v)

        for p in range(PRE):
            pltpu.async_copy(table_hbm.at[idx_v.at[p]], bufs[p], gsem)

        def outer(jo, _):
            for b in range(NBUF):
                j = jo * NBUF + b
                jn = j + PRE
                bn = (b + PRE) % NBUF

                @pl.when(jn < steps)
                def _fire():
                    @pl.when(jn >= NBUF)
                    def _drain():
                        pltpu.make_async_copy(
                            bufs[bn], out_hbm.at[pl.ds(base, CHUNK)], osem
                        ).wait()

                    pltpu.async_copy(table_hbm.at[idx_v.at[jn]], bufs[bn], gsem)

                pltpu.make_async_copy(
                    table_hbm.at[idx_v.at[0]], bufs[b], gsem
                ).wait()
                pltpu.async_copy(
                    bufs[b], out_hbm.at[pl.ds(base + j * CHUNK, CHUNK)], osem
                )
            return 0

        lax.fori_loop(0, steps // NBUF, outer, 0)
        for b in range(NBUF):
            pltpu.make_async_copy(
                bufs[b], out_hbm.at[pl.ds(base, CHUNK)], osem
            ).wait()

    return emb_kernel, num_workers, steps


def kernel(x, weight):
    batch, hist = x.shape
    n_flat = batch * hist
    emb, num_workers, steps = _make_kernel(n_flat)
    idx = x.reshape(num_workers, steps, CHUNK)
    out = emb(idx, weight)
    return out.reshape(batch, hist, EMBED_DIM)
